# baseline (device time: 24115 ns/iter reference)
import jax
import jax.numpy as jnp
from jax import lax
from jax.experimental import pallas as pl
from jax.experimental.pallas import tpu as pltpu


def kernel(Q, K, V):
    b, s, h, d = Q.shape
    bh = b * h
    hq = bh // 4
    hc = hq // 2
    scale = d ** -0.5

    Kt = K.transpose(0, 2, 3, 1).reshape(bh, d, s)
    Vt = V.transpose(0, 2, 3, 1).reshape(bh, d, s)

    def body(q_hbm, k_hbm, v_hbm, o_hbm, qv, kv, vv, rk, rv, ov,
             lsem, osem, send_sems, recv_sems):
        my_x = lax.axis_index("x")
        my_y = lax.axis_index("y")
        my_z = lax.axis_index("z")
        peer_x = (1 - my_x, my_y, my_z)
        nb_y = (my_x, 1 - my_y, my_z)
        nb_z = (my_x, my_y, 1 - my_z)

        j_me = 2 * my_y + my_z
        j_y = 2 * (1 - my_y) + my_z
        j_z = 2 * my_y + (1 - my_z)
        j_diag = 2 * (1 - my_y) + (1 - my_z)

        def csl(ref, j, c):
            return ref.at[pl.ds(hq * j + hc * c, hc)]

        cqs = [pltpu.make_async_copy(
                   q_hbm.at[i // h, :, i % h, :], qv.at[i], lsem.at[i])
               for i in range(bh)]
        ck = pltpu.make_async_copy(k_hbm, kv, lsem.at[bh])
        cv = pltpu.make_async_copy(v_hbm, vv, lsem.at[bh + 1])
        for c_ in cqs:
            c_.start()
        ck.start()
        cv.start()

        barrier = pltpu.get_barrier_semaphore()
        for nbr in (peer_x, nb_y, nb_z):
            pl.semaphore_signal(barrier, inc=1, device_id=nbr,
                                device_id_type=pl.DeviceIdType.MESH)
        pl.semaphore_wait(barrier, 3)

        def copy(src, dst, sem_i, dev):
            return pltpu.make_async_remote_copy(
                src_ref=src, dst_ref=dst,
                send_sem=send_sems.at[sem_i], recv_sem=recv_sems.at[sem_i],
                device_id=dev, device_id_type=pl.DeviceIdType.MESH,
            )

        C = (0, 1)
        o1k = [copy(csl(k_hbm, j_me, c), csl(rk, j_me, c), 0 + c, peer_x)
               for c in C]
        o1v = [copy(csl(v_hbm, j_me, c), csl(rv, j_me, c), 2 + c, peer_x)
               for c in C]
        dk = pl.ds(hq * j_diag, 1)
        dv = pl.ds(hq * j_diag, 1)
        o1dk = copy(k_hbm.at[dk], rk.at[dk], 16, peer_x)
        o1dv = copy(v_hbm.at[dv], rv.at[dv], 17, peer_x)
        o2ky = [copy(csl(rk, j_me, c), csl(rk, j_me, c), 4 + c, nb_y)
                for c in C]
        o2vy = [copy(csl(rv, j_me, c), csl(rv, j_me, c), 6 + c, nb_y)
                for c in C]
        o2kz = [copy(csl(rk, j_me, c), csl(rk, j_me, c), 8 + c, nb_z)
                for c in C]
        o2vz = [copy(csl(rv, j_me, c), csl(rv, j_me, c), 10 + c, nb_z)
                for c in C]
        def tail3(ref, j, c):
            return ref.at[pl.ds(hq * j + 1 + c, 1 + c)]

        o3y = [copy(tail3(rk, j_z, c), tail3(rk, j_z, c), 12 + c, nb_y)
               for c in C]
        o3z = [copy(tail3(rv, j_y, c), tail3(rv, j_y, c), 14 + c, nb_z)
               for c in C]

        dummy = kv.at[pl.ds(0, hc)]
        dummy1 = kv.at[pl.ds(0, 1)]
        i2k = [copy(dummy, csl(rk, j_y, c), 4 + c, nb_y) for c in C]
        i2v = [copy(dummy, csl(rv, j_y, c), 6 + c, nb_y) for c in C]
        i3k = [copy(dummy, csl(rk, j_z, c), 8 + c, nb_z) for c in C]
        i3v = [copy(dummy, csl(rv, j_z, c), 10 + c, nb_z) for c in C]
        i4k = [copy(kv.at[pl.ds(0, 1 + c)], tail3(rk, j_diag, c), 12 + c,
                    nb_y) for c in C]
        i4v = [copy(kv.at[pl.ds(0, 1 + c)], tail3(rv, j_diag, c), 14 + c,
                    nb_z) for c in C]
        i1dk = copy(dummy1, rk.at[pl.ds(hq * j_diag, 1)], 16, peer_x)
        i1dv = copy(dummy1, rv.at[pl.ds(hq * j_diag, 1)], 17, peer_x)

        def local_half(i):
            q = qv[i]
            s_l = lax.dot_general(
                q, kv[i], (((1,), (0,)), ((), ()))) * scale
            m_l = jnp.max(s_l, axis=1, keepdims=True)
            p_l = jnp.exp(s_l - m_l)
            l_l = jnp.sum(p_l, axis=1, keepdims=True)
            acc_l = lax.dot_general(p_l, vv[i], (((1,), (1,)), ((), ())))
            return m_l, l_l, acc_l

        def merge_quarter(j, parts):
            for t in range(hq):
                i = hq * j + t
                m_l, l_l, acc_l = parts[t]
                q = qv[i]
                s_r = lax.dot_general(
                    q, rk[i], (((1,), (0,)), ((), ()))) * scale
                m_r = jnp.max(s_r, axis=1, keepdims=True)
                m = jnp.maximum(m_l, m_r)
                p_r = jnp.exp(s_r - m)
                a_l = jnp.exp(m_l - m)
                denom = l_l * a_l + jnp.sum(p_r, axis=1, keepdims=True)
                acc = (acc_l * a_l
                       + lax.dot_general(p_r, rv[i], (((1,), (1,)), ((), ()))))
                ov[i] = acc / denom

        def writeback(j):
            for const in range(4):
                @pl.when(j == const)
                def _():
                    for t in range(hq):
                        i = hq * const + t
                        pltpu.make_async_copy(
                            ov.at[i], o_hbm.at[i // h, :, i % h, :],
                            osem.at[i]).start()

        o1k[0].start()
        o1v[0].start()
        o1k[1].start()
        o1v[1].start()
        o1dk.start()
        o1dv.start()
        for c_ in cqs:
            c_.wait()
        ck.wait()
        cv.wait()
        parts = {}
        parts[0] = [local_half(hq * j_me + t) for t in range(hq)]
        parts[1] = [local_half(hq * j_y + t) for t in range(hq)]
        o1k[0].wait_recv()
        o2ky[0].start()
        o2kz[0].start()
        parts[2] = [local_half(hq * j_z + t) for t in range(hq)]
        o1v[0].wait_recv()
        o2vy[0].start()
        o2vz[0].start()
        parts[3] = [local_half(hq * j_diag + t) for t in range(hq)]
        o1k[1].wait_recv()
        o2ky[1].start()
        o2kz[1].start()
        o1v[1].wait_recv()
        o2vy[1].start()
        o2vz[1].start()
        merge_quarter(j_me, parts[0])
        writeback(j_me)
        i3k[0].wait_recv()
        o3y[0].start()
        i2v[0].wait_recv()
        o3z[0].start()
        i3k[1].wait_recv()
        o3y[1].start()
        i2v[1].wait_recv()
        o3z[1].start()
        i2k[0].wait_recv()
        i2k[1].wait_recv()
        merge_quarter(j_y, parts[1])
        writeback(j_y)
        i3v[0].wait_recv()
        i3v[1].wait_recv()
        merge_quarter(j_z, parts[2])
        writeback(j_z)
        i1dk.wait_recv()
        i1dv.wait_recv()
        for c in C:
            i4k[c].wait_recv()
            i4v[c].wait_recv()
        merge_quarter(j_diag, parts[3])
        writeback(j_diag)

        for dsc in (o1k + o1v + o2ky + o2vy + o2kz + o2vz + o3y + o3z
                    + [o1dk, o1dv]):
            dsc.wait_send()
        for i in range(bh):
            pltpu.make_async_copy(
                ov.at[i], o_hbm.at[i // h, :, i % h, :], osem.at[i]).wait()

    out = pl.pallas_call(
        body,
        out_shape=jax.ShapeDtypeStruct((b, s, h, d), jnp.float32),
        in_specs=[pl.BlockSpec(memory_space=pltpu.MemorySpace.HBM)] * 3,
        out_specs=pl.BlockSpec(memory_space=pltpu.MemorySpace.HBM),
        scratch_shapes=[
            pltpu.VMEM((bh, s, d), jnp.float32),
            pltpu.VMEM((bh, d, s), jnp.float32),
            pltpu.VMEM((bh, d, s), jnp.float32),
            pltpu.VMEM((bh, d, s), jnp.float32),
            pltpu.VMEM((bh, d, s), jnp.float32),
            pltpu.VMEM((bh, s, d), jnp.float32),
            pltpu.SemaphoreType.DMA((bh + 2,)),
            pltpu.SemaphoreType.DMA((bh,)),
            pltpu.SemaphoreType.DMA((18,)),
            pltpu.SemaphoreType.DMA((18,)),
        ],
        compiler_params=pltpu.CompilerParams(collective_id=0),
    )(Q, Kt, Vt)

    return out


# device time: 21951 ns/iter; 1.0986x vs baseline; 1.0986x over previous
import jax
import jax.numpy as jnp
from jax import lax
from jax.experimental import pallas as pl
from jax.experimental.pallas import tpu as pltpu


def kernel(Q, K, V):
    b, s, h, d = Q.shape
    bh = b * h
    hq = bh // 4
    hc = hq // 2
    scale = d ** -0.5

    Qt = Q.transpose(0, 2, 1, 3).reshape(bh, s, d)
    Kt = K.transpose(0, 2, 3, 1).reshape(bh, d, s)
    Vt = V.transpose(0, 2, 3, 1).reshape(bh, d, s)

    def body(q_hbm, k_hbm, v_hbm, o_ref, qv, kv, vv, rk, rv,
             lsem, send_sems, recv_sems):
        my_x = lax.axis_index("x")
        my_y = lax.axis_index("y")
        my_z = lax.axis_index("z")
        peer_x = (1 - my_x, my_y, my_z)
        nb_y = (my_x, 1 - my_y, my_z)
        nb_z = (my_x, my_y, 1 - my_z)

        j_me = 2 * my_y + my_z
        j_y = 2 * (1 - my_y) + my_z
        j_z = 2 * my_y + (1 - my_z)
        j_diag = 2 * (1 - my_y) + (1 - my_z)

        def csl(ref, j, c):
            return ref.at[pl.ds(hq * j + hc * c, hc)]

        cq = pltpu.make_async_copy(q_hbm, qv, lsem.at[0])
        ck = pltpu.make_async_copy(k_hbm, kv, lsem.at[1])
        cv = pltpu.make_async_copy(v_hbm, vv, lsem.at[2])
        cq.start()
        ck.start()
        cv.start()

        barrier = pltpu.get_barrier_semaphore()
        for nbr in (peer_x, nb_y, nb_z):
            pl.semaphore_signal(barrier, inc=1, device_id=nbr,
                                device_id_type=pl.DeviceIdType.MESH)
        pl.semaphore_wait(barrier, 3)

        def copy(src, dst, sem_i, dev):
            return pltpu.make_async_remote_copy(
                src_ref=src, dst_ref=dst,
                send_sem=send_sems.at[sem_i], recv_sem=recv_sems.at[sem_i],
                device_id=dev, device_id_type=pl.DeviceIdType.MESH,
            )

        C = (0, 1)
        o1k = [copy(csl(k_hbm, j_me, c), csl(rk, j_me, c), 0 + c, peer_x)
               for c in C]
        o1v = [copy(csl(v_hbm, j_me, c), csl(rv, j_me, c), 2 + c, peer_x)
               for c in C]
        dk = pl.ds(hq * j_diag, 1)
        dv = pl.ds(hq * j_diag, 1)
        o1dk = copy(k_hbm.at[dk], rk.at[dk], 16, peer_x)
        o1dv = copy(v_hbm.at[dv], rv.at[dv], 17, peer_x)
        o2ky = [copy(csl(rk, j_me, c), csl(rk, j_me, c), 4 + c, nb_y)
                for c in C]
        o2vy = [copy(csl(rv, j_me, c), csl(rv, j_me, c), 6 + c, nb_y)
                for c in C]
        o2kz = [copy(csl(rk, j_me, c), csl(rk, j_me, c), 8 + c, nb_z)
                for c in C]
        o2vz = [copy(csl(rv, j_me, c), csl(rv, j_me, c), 10 + c, nb_z)
                for c in C]
        def tail3(ref, j, c):
            return ref.at[pl.ds(hq * j + 1 + c, 1 + c)]

        o3y = [copy(tail3(rk, j_z, c), tail3(rk, j_z, c), 12 + c, nb_y)
               for c in C]
        o3z = [copy(tail3(rv, j_y, c), tail3(rv, j_y, c), 14 + c, nb_z)
               for c in C]

        dummy = kv.at[pl.ds(0, hc)]
        dummy1 = kv.at[pl.ds(0, 1)]
        i2k = [copy(dummy, csl(rk, j_y, c), 4 + c, nb_y) for c in C]
        i2v = [copy(dummy, csl(rv, j_y, c), 6 + c, nb_y) for c in C]
        i3k = [copy(dummy, csl(rk, j_z, c), 8 + c, nb_z) for c in C]
        i3v = [copy(dummy, csl(rv, j_z, c), 10 + c, nb_z) for c in C]
        i4k = [copy(kv.at[pl.ds(0, 1 + c)], tail3(rk, j_diag, c), 12 + c,
                    nb_y) for c in C]
        i4v = [copy(kv.at[pl.ds(0, 1 + c)], tail3(rv, j_diag, c), 14 + c,
                    nb_z) for c in C]
        i1dk = copy(dummy1, rk.at[pl.ds(hq * j_diag, 1)], 16, peer_x)
        i1dv = copy(dummy1, rv.at[pl.ds(hq * j_diag, 1)], 17, peer_x)

        def local_half(i):
            q = qv[i]
            s_l = lax.dot_general(
                q, kv[i], (((1,), (0,)), ((), ()))) * scale
            m_l = jnp.max(s_l, axis=1, keepdims=True)
            p_l = jnp.exp(s_l - m_l)
            l_l = jnp.sum(p_l, axis=1, keepdims=True)
            acc_l = lax.dot_general(p_l, vv[i], (((1,), (1,)), ((), ())))
            return m_l, l_l, acc_l

        def merge_quarter(j, parts):
            for t in range(hq):
                i = hq * j + t
                m_l, l_l, acc_l = parts[t]
                q = qv[i]
                s_r = lax.dot_general(
                    q, rk[i], (((1,), (0,)), ((), ()))) * scale
                m_r = jnp.max(s_r, axis=1, keepdims=True)
                m = jnp.maximum(m_l, m_r)
                p_r = jnp.exp(s_r - m)
                a_l = jnp.exp(m_l - m)
                denom = l_l * a_l + jnp.sum(p_r, axis=1, keepdims=True)
                acc = (acc_l * a_l
                       + lax.dot_general(p_r, rv[i], (((1,), (1,)), ((), ()))))
                o_ref[i] = acc / denom

        o1k[0].start()
        o1v[0].start()
        o1k[1].start()
        o1v[1].start()
        o1dk.start()
        o1dv.start()
        cq.wait()
        ck.wait()
        cv.wait()
        parts = {}
        parts[0] = [local_half(hq * j_me + t) for t in range(hq)]
        parts[1] = [local_half(hq * j_y + t) for t in range(hq)]
        o1k[0].wait_recv()
        o2ky[0].start()
        o2kz[0].start()
        parts[2] = [local_half(hq * j_z + t) for t in range(hq)]
        o1v[0].wait_recv()
        o2vy[0].start()
        o2vz[0].start()
        parts[3] = [local_half(hq * j_diag + t) for t in range(hq)]
        o1k[1].wait_recv()
        o2ky[1].start()
        o2kz[1].start()
        o1v[1].wait_recv()
        o2vy[1].start()
        o2vz[1].start()
        merge_quarter(j_me, parts[0])
        i3k[0].wait_recv()
        o3y[0].start()
        i2v[0].wait_recv()
        o3z[0].start()
        i3k[1].wait_recv()
        o3y[1].start()
        i2v[1].wait_recv()
        o3z[1].start()
        i2k[0].wait_recv()
        i2k[1].wait_recv()
        merge_quarter(j_y, parts[1])
        i3v[0].wait_recv()
        i3v[1].wait_recv()
        merge_quarter(j_z, parts[2])
        i1dk.wait_recv()
        i1dv.wait_recv()
        for c in C:
            i4k[c].wait_recv()
            i4v[c].wait_recv()
        merge_quarter(j_diag, parts[3])

        for dsc in (o1k + o1v + o2ky + o2vy + o2kz + o2vz + o3y + o3z
                    + [o1dk, o1dv]):
            dsc.wait_send()

    out = pl.pallas_call(
        body,
        out_shape=jax.ShapeDtypeStruct((bh, s, d), jnp.float32),
        in_specs=[pl.BlockSpec(memory_space=pltpu.MemorySpace.HBM)] * 3,
        out_specs=pl.BlockSpec(memory_space=pltpu.MemorySpace.VMEM),
        scratch_shapes=[
            pltpu.VMEM((bh, s, d), jnp.float32),
            pltpu.VMEM((bh, d, s), jnp.float32),
            pltpu.VMEM((bh, d, s), jnp.float32),
            pltpu.VMEM((bh, d, s), jnp.float32),
            pltpu.VMEM((bh, d, s), jnp.float32),
            pltpu.SemaphoreType.DMA((3,)),
            pltpu.SemaphoreType.DMA((18,)),
            pltpu.SemaphoreType.DMA((18,)),
        ],
        compiler_params=pltpu.CompilerParams(collective_id=0),
    )(Qt, Kt, Vt)

    return out.reshape(b, h, s, d).transpose(0, 2, 1, 3)
